# Initial kernel scaffold; baseline (speedup 1.0000x reference)
import jax
import jax.numpy as jnp
from jax.experimental import pallas as pl


def kernel(x_tasks, x_data, ea_dt, ea_td, ea_tt, params, ei_dt, ei_td, ei_tt):
    raise NotImplementedError("write your pallas kernel here")


# speedup vs baseline: 1.7895x; 1.7895x over previous
import functools

import jax
import jax.numpy as jnp
from jax import lax
from jax.experimental import pallas as pl

HID = 64
NDEV = 5


def _leaky(x, s):
    return jnp.where(x > 0, x, s * x)


def _ln(x, g, b, eps=1e-5):
    m = x.mean(-1, keepdims=True)
    v = ((x - m) ** 2).mean(-1, keepdims=True)
    return (x - m) / jnp.sqrt(v + eps) * g + b


def _gat(x_src, x_dst, ei, ae, p):
    src, dst = ei[0], ei[1]
    n = x_dst.shape[0]
    hs = x_src @ p['W_src']
    a_s = hs @ p['att_src'][0]
    a_d = x_dst @ (p['W_dst'] @ p['att_dst'][0])
    a = a_s[src] + a_d[dst] + ae
    a = _leaky(a, 0.2)
    ex = jnp.exp(a)
    den = jax.ops.segment_sum(ex, dst, num_segments=n)
    num = jax.ops.segment_sum(hs[src] * ex[:, None], dst, num_segments=n)
    out = num / (den[:, None] + 1e-16)
    return out + x_dst @ p['W_res'] + p['b']


def _head2_body(z2_ref, r_ref, w1a_ref, b1_ref, g1_ref, bn1_ref,
                w2_ref, b2_ref, out_ref):
    h = jnp.dot(z2_ref[...], w1a_ref[...],
                preferred_element_type=jnp.float32) + r_ref[...] + b1_ref[...]
    h = _ln(h, g1_ref[...], bn1_ref[...])
    h = _leaky(h, 0.01)
    out_ref[...] = jnp.dot(h, w2_ref[...],
                           preferred_element_type=jnp.float32) + b2_ref[...]


def _head2(z2, r, w1a, b1, g1, bn1, w2p, b2p):
    n = z2.shape[0]
    blk = 512
    grid = n // blk
    return pl.pallas_call(
        _head2_body,
        grid=(grid,),
        in_specs=[
            pl.BlockSpec((blk, HID), lambda i: (i, 0)),
            pl.BlockSpec((1, HID), lambda i: (0, 0)),
            pl.BlockSpec((HID, HID), lambda i: (0, 0)),
            pl.BlockSpec((1, HID), lambda i: (0, 0)),
            pl.BlockSpec((1, HID), lambda i: (0, 0)),
            pl.BlockSpec((1, HID), lambda i: (0, 0)),
            pl.BlockSpec((HID, 128), lambda i: (0, 0)),
            pl.BlockSpec((1, 128), lambda i: (0, 0)),
        ],
        out_specs=pl.BlockSpec((blk, 128), lambda i: (i, 0)),
        out_shape=jax.ShapeDtypeStruct((n, 128), jnp.float32),
    )(z2, r, w1a, b1, g1, bn1, w2p, b2p)


def kernel(x_tasks, x_data, ea_dt, ea_td, ea_tt, params, ei_dt, ei_td, ei_tt):
    P = params
    act = lambda x: _leaky(x, 0.01)
    nt = x_tasks.shape[0]

    def ae_of(ea, p):
        return ea @ (p['W_edge'] @ p['att_edge'][0])

    t1 = _gat(x_data, x_tasks, ei_dt, ae_of(ea_dt, P['dt1']), P['dt1'])
    d1 = _gat(x_tasks, x_data, ei_td, ae_of(ea_td, P['td1']), P['td1'])
    t1 = _ln(act(t1), P['ln_t']['g'], P['ln_t']['b'])
    d1 = _ln(act(d1), P['ln_d']['g'], P['ln_d']['b'])
    t2 = _gat(d1, t1, ei_dt, ae_of(ea_dt, P['dt2']), P['dt2'])
    ei_flip = ei_tt[::-1]
    dep = _gat(t2, t2, ei_tt, ae_of(ea_tt, P['dep1']), P['dep1'])
    dep = act(_ln(dep, P['ln_dep']['g'], P['ln_dep']['b']))
    dpt = _gat(t2, t2, ei_flip, ae_of(ea_tt, P['dpt1']), P['dpt1'])
    dpt = act(_ln(dpt, P['ln_dpt']['g'], P['ln_dpt']['b']))
    dep = _gat(dep, dep, ei_tt, ae_of(ea_tt, P['dep2']), P['dep2'])
    dpt = _gat(dpt, dpt, ei_flip, ae_of(ea_tt, P['dpt2']), P['dpt2'])

    zx = act(_ln(dep @ P['fc_x']['W'][:HID] + dpt @ P['fc_x']['W'][HID:]
                 + P['fc_x']['b'], P['ln_x']['g'], P['ln_x']['b']))
    zy = act(_ln(t2 @ P['fc_y']['W'] + P['fc_y']['b'],
                 P['ln_y']['g'], P['ln_y']['b']))
    z = zx + zy
    z = act(_ln(z @ P['fc_c']['W'] + P['fc_c']['b'],
                P['ln_c']['g'], P['ln_c']['b']))

    zmean = z.mean(axis=0, keepdims=True)
    r = zmean @ P['fc1']['W'][HID:]
    npad = 50176
    zp = jnp.zeros((npad, HID), jnp.float32).at[:nt].set(z)
    w2p = jnp.zeros((HID, 128), jnp.float32).at[:, :NDEV].set(P['fc2']['W'])
    b2p = jnp.zeros((1, 128), jnp.float32).at[0, :NDEV].set(P['fc2']['b'])
    out = _head2(zp, r, P['fc1']['W'][:HID], P['fc1']['b'][None],
                 P['ln1']['g'][None], P['ln1']['b'][None], w2p, b2p)
    return out[:nt, :NDEV]


# speedup vs baseline: 11.8191x; 6.6048x over previous
import functools

import jax
import jax.numpy as jnp
from jax import lax
from jax.experimental import pallas as pl
from jax.experimental.pallas import tpu as pltpu
from jax.experimental.pallas import tpu_sc as plsc

HID = 64
W80 = 80
NDEV = 5
NT = 50000
NPAD = 50176
H2 = 25024
E = 800000
NSUB = 16
PER_SUB = 50176
EPAD = PER_SUB * NSUB
PER_A = EPAD // 32
CHA = 512
CHB = 64
NZCH = H2 // CHB


def _leaky(x, s):
    return jnp.where(x > 0, x, s * x)


def _ln(x, g, b, eps=1e-5):
    m = x.mean(-1, keepdims=True)
    v = ((x - m) ** 2).mean(-1, keepdims=True)
    return (x - m) / jnp.sqrt(v + eps) * g + b


def _sc_logit_body(src_hbm, dst_hbm, ae_hbm, as_hbm, ad_hbm, ex_out,
                   as_v, ad_v, srcb, dstb, aeb, exb):
    c = lax.axis_index("c")
    s = lax.axis_index("s")
    w = s * 2 + c
    pltpu.sync_copy(as_hbm, as_v)
    pltpu.sync_copy(ad_hbm, ad_v)

    def chunk(g, _):
        off = w * PER_A + g * CHA
        pltpu.sync_copy(src_hbm.at[pl.ds(off, CHA)], srcb)
        pltpu.sync_copy(dst_hbm.at[pl.ds(off, CHA)], dstb)
        pltpu.sync_copy(ae_hbm.at[pl.ds(off, CHA)], aeb)
        for j in range(CHA // 16):
            sv = srcb[pl.ds(j * 16, 16)]
            dv = dstb[pl.ds(j * 16, 16)]
            a = (plsc.load_gather(as_v, [sv]) + plsc.load_gather(ad_v, [dv])
                 + aeb[pl.ds(j * 16, 16)])
            a = jnp.where(a > 0, a, a * jnp.float32(0.2))
            exb[pl.ds(j * 16, 16)] = jnp.exp(a)
        pltpu.sync_copy(exb, ex_out.at[pl.ds(off, CHA)])
        return 0
    lax.fori_loop(0, PER_A // CHA, chunk, 0)


_sc_logit = functools.partial(
    pl.kernel,
    out_type=jax.ShapeDtypeStruct((EPAD,), jnp.float32),
    mesh=plsc.VectorSubcoreMesh(core_axis_name="c", subcore_axis_name="s"),
    compiler_params=pltpu.CompilerParams(needs_layout_passes=False,
                                         use_tc_tiling_on_sc=False),
    scratch_types=[
        pltpu.VMEM((NPAD,), jnp.float32),
        pltpu.VMEM((NPAD,), jnp.float32),
        pltpu.VMEM((CHA,), jnp.int32),
        pltpu.VMEM((CHA,), jnp.int32),
        pltpu.VMEM((CHA,), jnp.float32),
        pltpu.VMEM((CHA,), jnp.float32),
    ],
)(_sc_logit_body)


def _sc_agg_body(src_hbm, dst_hbm, ex_hbm, hs_hbm, acc_out,
                 srcb, dstb, exb, idxb, rows, acc_sh, sem):
    c = lax.axis_index("c")
    s = lax.axis_index("s")
    base = c * H2
    zf = jnp.zeros((16,), jnp.float32)

    def zrow(i, _):
        for k in range(W80 // 16):
            rows[i, pl.ds(k * 16, 16)] = zf
        return 0
    lax.fori_loop(0, CHB, zrow, 0)

    def zfill(i, _):
        t = i * NSUB + s
        @pl.when(t < NZCH)
        def _():
            pltpu.sync_copy(rows, acc_sh.at[pl.ds(t * CHB, CHB)])
        return 0
    lax.fori_loop(0, NZCH // NSUB + 1, zfill, 0)
    plsc.subcore_barrier()

    def chunk(g, _):
        off = s * PER_SUB + g * CHB
        pltpu.sync_copy(src_hbm.at[pl.ds(off, CHB)], srcb)
        pltpu.sync_copy(dst_hbm.at[pl.ds(off, CHB)], dstb)
        pltpu.sync_copy(ex_hbm.at[pl.ds(off, CHB)], exb)
        pltpu.async_copy(hs_hbm.at[srcb], rows, sem).wait()
        for j in range(CHB // 16):
            dv = dstb[pl.ds(j * 16, 16)]
            ex = exb[pl.ds(j * 16, 16)]
            own = (dv >= base) & (dv < base + H2)
            exm = jnp.where(own, ex, jnp.float32(0.0))
            idxb[pl.ds(j * 16, 16)] = jnp.where(own, dv - base, 0)
            for l in range(16):
                sx = exm[l]
                e = j * 16 + l
                for k in range(W80 // 16):
                    rows[e, pl.ds(k * 16, 16)] = (
                        rows[e, pl.ds(k * 16, 16)] * sx)
        pltpu.sync_copy(rows, acc_sh.at[idxb], add=True)
        return 0
    lax.fori_loop(0, PER_SUB // CHB, chunk, 0)
    plsc.subcore_barrier()

    def wb(i, _):
        t = i * NSUB + s
        @pl.when(t < NZCH)
        def _():
            pltpu.sync_copy(acc_sh.at[pl.ds(t * CHB, CHB)], rows)
            pltpu.sync_copy(rows, acc_out.at[c, pl.ds(t * CHB, CHB)])
        return 0
    lax.fori_loop(0, NZCH // NSUB + 1, wb, 0)


_sc_agg = functools.partial(
    pl.kernel,
    out_type=jax.ShapeDtypeStruct((2, H2, W80), jnp.float32),
    mesh=plsc.VectorSubcoreMesh(core_axis_name="c", subcore_axis_name="s"),
    compiler_params=pltpu.CompilerParams(needs_layout_passes=False,
                                         use_tc_tiling_on_sc=False),
    scratch_types=[
        pltpu.VMEM((CHB,), jnp.int32),
        pltpu.VMEM((CHB,), jnp.int32),
        pltpu.VMEM((CHB,), jnp.float32),
        pltpu.VMEM((CHB,), jnp.int32),
        pltpu.VMEM((CHB, W80), jnp.float32),
        pltpu.VMEM_SHARED((H2, W80), jnp.float32),
        pltpu.SemaphoreType.DMA,
    ],
)(_sc_agg_body)


def _gat_sc(hs80, a_s, a_d, edges, ae, x_dst, W_res, b):
    ex = _sc_logit(edges[0], edges[1], ae, a_s, a_d)
    acc = _sc_agg(edges[0], edges[1], ex, hs80)
    acc = acc.reshape(2 * H2, W80)
    num = acc[:NT, :HID]
    den = acc[:NT, HID]
    return num / (den[:, None] + 1e-16) + x_dst @ W_res + b


def _padn(x):
    return jnp.zeros((NPAD,) + x.shape[1:], x.dtype).at[:x.shape[0]].set(x)


def _pade(x, fill):
    return jnp.concatenate(
        [x, jnp.full((EPAD - E,) + x.shape[1:], fill, x.dtype)])


def _prep(x_src, x_dst, p):
    hs = x_src @ p['W_src']
    hs80 = jnp.zeros((NPAD, W80), jnp.float32)
    hs80 = hs80.at[:hs.shape[0], :HID].set(hs)
    hs80 = hs80.at[:, HID].set(1.0)
    a_s = hs @ p['att_src'][0]
    a_d = x_dst @ (p['W_dst'] @ p['att_dst'][0])
    return hs80, _padn(a_s), _padn(a_d)


def _ae_of(ea, p):
    return _pade(ea @ (p['W_edge'] @ p['att_edge'][0]), 0.0)


def _head2_body(z2_ref, r_ref, w1a_ref, b1_ref, g1_ref, bn1_ref,
                w2_ref, b2_ref, out_ref):
    h = jnp.dot(z2_ref[...], w1a_ref[...],
                preferred_element_type=jnp.float32) + r_ref[...] + b1_ref[...]
    h = _ln(h, g1_ref[...], bn1_ref[...])
    h = _leaky(h, 0.01)
    out_ref[...] = jnp.dot(h, w2_ref[...],
                           preferred_element_type=jnp.float32) + b2_ref[...]


def _head2(z2, r, w1a, b1, g1, bn1, w2p, b2p):
    n = z2.shape[0]
    blk = 512
    return pl.pallas_call(
        _head2_body,
        grid=(n // blk,),
        in_specs=[
            pl.BlockSpec((blk, HID), lambda i: (i, 0)),
            pl.BlockSpec((1, HID), lambda i: (0, 0)),
            pl.BlockSpec((HID, HID), lambda i: (0, 0)),
            pl.BlockSpec((1, HID), lambda i: (0, 0)),
            pl.BlockSpec((1, HID), lambda i: (0, 0)),
            pl.BlockSpec((1, HID), lambda i: (0, 0)),
            pl.BlockSpec((HID, 128), lambda i: (0, 0)),
            pl.BlockSpec((1, 128), lambda i: (0, 0)),
        ],
        out_specs=pl.BlockSpec((blk, 128), lambda i: (i, 0)),
        out_shape=jax.ShapeDtypeStruct((n, 128), jnp.float32),
    )(z2, r, w1a, b1, g1, bn1, w2p, b2p)


def kernel(x_tasks, x_data, ea_dt, ea_td, ea_tt, params, ei_dt, ei_td, ei_tt):
    P = params
    act = lambda x: _leaky(x, 0.01)

    e_dt = (_pade(ei_dt[0], 0), _pade(ei_dt[1], NT))
    e_td = (_pade(ei_td[0], 0), _pade(ei_td[1], NT))
    e_tt = (_pade(ei_tt[0], 0), _pade(ei_tt[1], NT))
    e_ttf = (e_tt[1], _pade(ei_tt[0], NT))

    hs80, a_s, a_d = _prep(x_data, x_tasks, P['dt1'])
    t1 = _gat_sc(hs80, a_s, a_d, e_dt, _ae_of(ea_dt, P['dt1']),
                 x_tasks, P['dt1']['W_res'], P['dt1']['b'])
    hs80, a_s, a_d = _prep(x_tasks, x_data, P['td1'])
    d1 = _gat_sc(hs80, a_s, a_d, e_td, _ae_of(ea_td, P['td1']),
                 x_data, P['td1']['W_res'], P['td1']['b'])
    t1 = _ln(act(t1), P['ln_t']['g'], P['ln_t']['b'])
    d1 = _ln(act(d1), P['ln_d']['g'], P['ln_d']['b'])

    hs80, a_s, a_d = _prep(d1, t1, P['dt2'])
    t2 = _gat_sc(hs80, a_s, a_d, e_dt, _ae_of(ea_dt, P['dt2']),
                 t1, P['dt2']['W_res'], P['dt2']['b'])

    hs80, a_s, a_d = _prep(t2, t2, P['dep1'])
    dep = _gat_sc(hs80, a_s, a_d, e_tt, _ae_of(ea_tt, P['dep1']),
                  t2, P['dep1']['W_res'], P['dep1']['b'])
    dep = act(_ln(dep, P['ln_dep']['g'], P['ln_dep']['b']))
    hs80, a_s, a_d = _prep(t2, t2, P['dpt1'])
    dpt = _gat_sc(hs80, a_s, a_d, e_ttf, _ae_of(ea_tt, P['dpt1']),
                  t2, P['dpt1']['W_res'], P['dpt1']['b'])
    dpt = act(_ln(dpt, P['ln_dpt']['g'], P['ln_dpt']['b']))

    hs80, a_s, a_d = _prep(dep, dep, P['dep2'])
    dep = _gat_sc(hs80, a_s, a_d, e_tt, _ae_of(ea_tt, P['dep2']),
                  dep, P['dep2']['W_res'], P['dep2']['b'])
    hs80, a_s, a_d = _prep(dpt, dpt, P['dpt2'])
    dpt = _gat_sc(hs80, a_s, a_d, e_ttf, _ae_of(ea_tt, P['dpt2']),
                  dpt, P['dpt2']['W_res'], P['dpt2']['b'])

    zx = act(_ln(dep @ P['fc_x']['W'][:HID] + dpt @ P['fc_x']['W'][HID:]
                 + P['fc_x']['b'], P['ln_x']['g'], P['ln_x']['b']))
    zy = act(_ln(t2 @ P['fc_y']['W'] + P['fc_y']['b'],
                 P['ln_y']['g'], P['ln_y']['b']))
    z = zx + zy
    z = act(_ln(z @ P['fc_c']['W'] + P['fc_c']['b'],
                P['ln_c']['g'], P['ln_c']['b']))

    zmean = z.mean(axis=0, keepdims=True)
    r = zmean @ P['fc1']['W'][HID:]
    zp = _padn(z)
    w2p = jnp.zeros((HID, 128), jnp.float32).at[:, :NDEV].set(P['fc2']['W'])
    b2p = jnp.zeros((1, 128), jnp.float32).at[0, :NDEV].set(P['fc2']['b'])
    out = _head2(zp, r, P['fc1']['W'][:HID], P['fc1']['b'][None],
                 P['ln1']['g'][None], P['ln1']['b'][None], w2p, b2p)
    return out[:NT, :NDEV]


# speedup vs baseline: 22.7535x; 1.9252x over previous
import functools

import jax
import jax.numpy as jnp
from jax import lax
from jax.experimental import pallas as pl
from jax.experimental.pallas import tpu as pltpu
from jax.experimental.pallas import tpu_sc as plsc

HID = 64
W80 = 80
NDEV = 5
NT = 50000
NPAD = 50176
H2 = NPAD // 2
E = 800000
NSUB = 16
PER_SUB = 50176
EPAD = PER_SUB * NSUB
EPAD4 = EPAD * 4
PER_A = EPAD // 32
CHA = 512
CHB = 32
NCHB = PER_SUB // CHB
NZCH = H2 // CHB


def _leaky(x, s):
    return jnp.where(x > 0, x, s * x)


def _ln(x, g, b, eps=1e-5):
    m = x.mean(-1, keepdims=True)
    v = ((x - m) ** 2).mean(-1, keepdims=True)
    return (x - m) / jnp.sqrt(v + eps) * g + b


def _sc_logit_body(src_hbm, dst_hbm, ae_hbm, as_hbm, ad_hbm, ed_out,
                   as_v, ad_v, srcb, dstb, aeb, edb):
    c = lax.axis_index("c")
    s = lax.axis_index("s")
    w = s * 2 + c
    iot = lax.iota(jnp.int32, 16)
    pltpu.sync_copy(as_hbm, as_v)
    pltpu.sync_copy(ad_hbm, ad_v)

    def chunk(g, _):
        off = w * PER_A + g * CHA
        pltpu.sync_copy(src_hbm.at[pl.ds(off, CHA)], srcb)
        pltpu.sync_copy(dst_hbm.at[pl.ds(off, CHA)], dstb)
        pltpu.sync_copy(ae_hbm.at[pl.ds(off, CHA)], aeb)
        for j in range(CHA // 16):
            sv = srcb[pl.ds(j * 16, 16)]
            dv = dstb[pl.ds(j * 16, 16)]
            a = (plsc.load_gather(as_v, [sv]) + plsc.load_gather(ad_v, [dv])
                 + aeb[pl.ds(j * 16, 16)])
            a = jnp.where(a > 0, a, a * jnp.float32(0.2))
            exi = plsc.bitcast(jnp.exp(a), jnp.int32)
            pos = iot * 4 + j * 64
            plsc.store_scatter(edb, [pos], sv)
            plsc.store_scatter(edb, [pos + 1], dv)
            plsc.store_scatter(edb, [pos + 2], exi)
        pltpu.sync_copy(edb, ed_out.at[pl.ds(off * 4, CHA * 4)])
        return 0
    lax.fori_loop(0, PER_A // CHA, chunk, 0)


_sc_logit = functools.partial(
    pl.kernel,
    out_type=jax.ShapeDtypeStruct((EPAD4,), jnp.int32),
    mesh=plsc.VectorSubcoreMesh(core_axis_name="c", subcore_axis_name="s"),
    compiler_params=pltpu.CompilerParams(needs_layout_passes=False,
                                         use_tc_tiling_on_sc=False),
    scratch_types=[
        pltpu.VMEM((NPAD,), jnp.float32),
        pltpu.VMEM((NPAD,), jnp.float32),
        pltpu.VMEM((CHA,), jnp.int32),
        pltpu.VMEM((CHA,), jnp.int32),
        pltpu.VMEM((CHA,), jnp.float32),
        pltpu.VMEM((CHA * 4,), jnp.int32),
    ],
)(_sc_logit_body)


def _sc_agg_body(ed_hbm, hs_hbm, acc_out,
                 edata, srcb, idxb, exmb, rows, acc_sh,
                 sem_l, sem_g, sem_s):
    c = lax.axis_index("c")
    s = lax.axis_index("s")
    base = c * H2
    iot = lax.iota(jnp.int32, 16)
    zf = jnp.zeros((16,), jnp.float32)
    ebase = s * PER_SUB * 4

    def zrow(i, _):
        for b in range(2):
            for k in range(W80 // 16):
                rows[b, i, pl.ds(k * 16, 16)] = zf
        return 0
    lax.fori_loop(0, CHB, zrow, 0)

    def zfill(i, _):
        t = i * NSUB + s
        pltpu.sync_copy(rows.at[0], acc_sh.at[pl.ds(t * CHB, CHB)])
        return 0
    lax.fori_loop(0, NZCH // NSUB, zfill, 0)
    plsc.subcore_barrier()

    def lin_start(g, b):
        pltpu.async_copy(ed_hbm.at[pl.ds(ebase + g * CHB * 4, CHB * 4)],
                         edata.at[b], sem_l.at[b])

    def lin_wait(g, b):
        pltpu.make_async_copy(ed_hbm.at[pl.ds(ebase + g * CHB * 4, CHB * 4)],
                              edata.at[b], sem_l.at[b]).wait()

    def gath_start(b):
        pltpu.async_copy(hs_hbm.at[srcb.at[b]], rows.at[b], sem_g.at[b])

    def gath_wait(b):
        pltpu.make_async_copy(hs_hbm.at[srcb.at[b]], rows.at[b],
                              sem_g.at[b]).wait()

    def scat_start(b):
        pltpu.async_copy(rows.at[b], acc_sh.at[idxb.at[b]], sem_s.at[b],
                         add=True)

    def scat_wait(b):
        pltpu.make_async_copy(rows.at[b], acc_sh.at[idxb.at[b]],
                              sem_s.at[b]).wait()

    def decode(b):
        ed = edata.at[b]
        for j in range(CHB // 16):
            pos = iot * 4 + j * 64
            sv = plsc.load_gather(ed, [pos])
            dv = plsc.load_gather(ed, [pos + 1])
            exv = plsc.bitcast(plsc.load_gather(ed, [pos + 2]), jnp.float32)
            own = (dv >= base) & (dv < base + H2)
            srcb[b, pl.ds(j * 16, 16)] = sv
            idxb[b, pl.ds(j * 16, 16)] = jnp.where(own, dv - base, 0)
            exmb[b, pl.ds(j * 16, 16)] = jnp.where(own, exv, jnp.float32(0.0))

    def scale(b):
        for j in range(CHB // 16):
            exm = exmb[b, pl.ds(j * 16, 16)]
            for l in range(16):
                sx = exm[l]
                e = j * 16 + l
                for k in range(W80 // 16):
                    rows[b, e, pl.ds(k * 16, 16)] = (
                        rows[b, e, pl.ds(k * 16, 16)] * sx)

    lin_start(0, 0)

    def outer(g2, _):
        for b in range(2):
            g = g2 * 2 + b
            lin_wait(g, b)
            @pl.when(g >= 2)
            def _():
                scat_wait(b)
            decode(b)
            gath_start(b)
            @pl.when(g + 1 < NCHB)
            def _():
                lin_start(g + 1, 1 - b)
            @pl.when(g >= 1)
            def _():
                gath_wait(1 - b)
                scale(1 - b)
                scat_start(1 - b)
        return 0
    lax.fori_loop(0, NCHB // 2, outer, 0)
    gath_wait(1)
    scale(1)
    scat_start(1)
    scat_wait(0)
    scat_wait(1)
    plsc.subcore_barrier()

    def wb(i, _):
        t = i * NSUB + s
        pltpu.sync_copy(acc_sh.at[pl.ds(t * CHB, CHB)], rows.at[0])
        pltpu.sync_copy(rows.at[0], acc_out.at[c, pl.ds(t * CHB, CHB)])
        return 0
    lax.fori_loop(0, NZCH // NSUB, wb, 0)


_sc_agg = functools.partial(
    pl.kernel,
    out_type=jax.ShapeDtypeStruct((2, H2, W80), jnp.float32),
    mesh=plsc.VectorSubcoreMesh(core_axis_name="c", subcore_axis_name="s"),
    compiler_params=pltpu.CompilerParams(needs_layout_passes=False,
                                         use_tc_tiling_on_sc=False),
    scratch_types=[
        pltpu.VMEM((2, CHB * 4), jnp.int32),
        pltpu.VMEM((2, CHB), jnp.int32),
        pltpu.VMEM((2, CHB), jnp.int32),
        pltpu.VMEM((2, CHB), jnp.float32),
        pltpu.VMEM((2, CHB, W80), jnp.float32),
        pltpu.VMEM_SHARED((H2, W80), jnp.float32),
        pltpu.SemaphoreType.DMA((2,)),
        pltpu.SemaphoreType.DMA((2,)),
        pltpu.SemaphoreType.DMA((2,)),
    ],
)(_sc_agg_body)


def _gat_sc(hs80, a_s, a_d, edges, ae, x_dst, W_res, b):
    ed = _sc_logit(edges[0], edges[1], ae, a_s, a_d)
    acc = _sc_agg(ed, hs80)
    acc = acc.reshape(2 * H2, W80)
    num = acc[:NT, :HID]
    den = acc[:NT, HID]
    return num / (den[:, None] + 1e-16) + x_dst @ W_res + b


def _padn(x):
    return jnp.zeros((NPAD,) + x.shape[1:], x.dtype).at[:x.shape[0]].set(x)


def _pade(x, fill):
    return jnp.concatenate(
        [x, jnp.full((EPAD - E,) + x.shape[1:], fill, x.dtype)])


def _prep(x_src, x_dst, p):
    hs = x_src @ p['W_src']
    hs80 = jnp.zeros((NPAD, W80), jnp.float32)
    hs80 = hs80.at[:hs.shape[0], :HID].set(hs)
    hs80 = hs80.at[:, HID].set(1.0)
    a_s = hs @ p['att_src'][0]
    a_d = x_dst @ (p['W_dst'] @ p['att_dst'][0])
    return hs80, _padn(a_s), _padn(a_d)


def _ae_of(ea, p):
    return _pade(ea @ (p['W_edge'] @ p['att_edge'][0]), 0.0)


def _head2_body(z2_ref, r_ref, w1a_ref, b1_ref, g1_ref, bn1_ref,
                w2_ref, b2_ref, out_ref):
    h = jnp.dot(z2_ref[...], w1a_ref[...],
                preferred_element_type=jnp.float32) + r_ref[...] + b1_ref[...]
    h = _ln(h, g1_ref[...], bn1_ref[...])
    h = _leaky(h, 0.01)
    out_ref[...] = jnp.dot(h, w2_ref[...],
                           preferred_element_type=jnp.float32) + b2_ref[...]


def _head2(z2, r, w1a, b1, g1, bn1, w2p, b2p):
    n = z2.shape[0]
    blk = 512
    return pl.pallas_call(
        _head2_body,
        grid=(n // blk,),
        in_specs=[
            pl.BlockSpec((blk, HID), lambda i: (i, 0)),
            pl.BlockSpec((1, HID), lambda i: (0, 0)),
            pl.BlockSpec((HID, HID), lambda i: (0, 0)),
            pl.BlockSpec((1, HID), lambda i: (0, 0)),
            pl.BlockSpec((1, HID), lambda i: (0, 0)),
            pl.BlockSpec((1, HID), lambda i: (0, 0)),
            pl.BlockSpec((HID, 128), lambda i: (0, 0)),
            pl.BlockSpec((1, 128), lambda i: (0, 0)),
        ],
        out_specs=pl.BlockSpec((blk, 128), lambda i: (i, 0)),
        out_shape=jax.ShapeDtypeStruct((n, 128), jnp.float32),
    )(z2, r, w1a, b1, g1, bn1, w2p, b2p)


def kernel(x_tasks, x_data, ea_dt, ea_td, ea_tt, params, ei_dt, ei_td, ei_tt):
    P = params
    act = lambda x: _leaky(x, 0.01)

    e_dt = (_pade(ei_dt[0], 0), _pade(ei_dt[1], NT))
    e_td = (_pade(ei_td[0], 0), _pade(ei_td[1], NT))
    e_tt = (_pade(ei_tt[0], 0), _pade(ei_tt[1], NT))
    e_ttf = (e_tt[1], _pade(ei_tt[0], NT))

    hs80, a_s, a_d = _prep(x_data, x_tasks, P['dt1'])
    t1 = _gat_sc(hs80, a_s, a_d, e_dt, _ae_of(ea_dt, P['dt1']),
                 x_tasks, P['dt1']['W_res'], P['dt1']['b'])
    hs80, a_s, a_d = _prep(x_tasks, x_data, P['td1'])
    d1 = _gat_sc(hs80, a_s, a_d, e_td, _ae_of(ea_td, P['td1']),
                 x_data, P['td1']['W_res'], P['td1']['b'])
    t1 = _ln(act(t1), P['ln_t']['g'], P['ln_t']['b'])
    d1 = _ln(act(d1), P['ln_d']['g'], P['ln_d']['b'])

    hs80, a_s, a_d = _prep(d1, t1, P['dt2'])
    t2 = _gat_sc(hs80, a_s, a_d, e_dt, _ae_of(ea_dt, P['dt2']),
                 t1, P['dt2']['W_res'], P['dt2']['b'])

    hs80, a_s, a_d = _prep(t2, t2, P['dep1'])
    dep = _gat_sc(hs80, a_s, a_d, e_tt, _ae_of(ea_tt, P['dep1']),
                  t2, P['dep1']['W_res'], P['dep1']['b'])
    dep = act(_ln(dep, P['ln_dep']['g'], P['ln_dep']['b']))
    hs80, a_s, a_d = _prep(t2, t2, P['dpt1'])
    dpt = _gat_sc(hs80, a_s, a_d, e_ttf, _ae_of(ea_tt, P['dpt1']),
                  t2, P['dpt1']['W_res'], P['dpt1']['b'])
    dpt = act(_ln(dpt, P['ln_dpt']['g'], P['ln_dpt']['b']))

    hs80, a_s, a_d = _prep(dep, dep, P['dep2'])
    dep = _gat_sc(hs80, a_s, a_d, e_tt, _ae_of(ea_tt, P['dep2']),
                  dep, P['dep2']['W_res'], P['dep2']['b'])
    hs80, a_s, a_d = _prep(dpt, dpt, P['dpt2'])
    dpt = _gat_sc(hs80, a_s, a_d, e_ttf, _ae_of(ea_tt, P['dpt2']),
                  dpt, P['dpt2']['W_res'], P['dpt2']['b'])

    zx = act(_ln(dep @ P['fc_x']['W'][:HID] + dpt @ P['fc_x']['W'][HID:]
                 + P['fc_x']['b'], P['ln_x']['g'], P['ln_x']['b']))
    zy = act(_ln(t2 @ P['fc_y']['W'] + P['fc_y']['b'],
                 P['ln_y']['g'], P['ln_y']['b']))
    z = zx + zy
    z = act(_ln(z @ P['fc_c']['W'] + P['fc_c']['b'],
                P['ln_c']['g'], P['ln_c']['b']))

    zmean = z.mean(axis=0, keepdims=True)
    r = zmean @ P['fc1']['W'][HID:]
    zp = _padn(z)
    w2p = jnp.zeros((HID, 128), jnp.float32).at[:, :NDEV].set(P['fc2']['W'])
    b2p = jnp.zeros((1, 128), jnp.float32).at[0, :NDEV].set(P['fc2']['b'])
    out = _head2(zp, r, P['fc1']['W'][:HID], P['fc1']['b'][None],
                 P['ln1']['g'][None], P['ln1']['b'][None], w2p, b2p)
    return out[:NT, :NDEV]
